# baseline (speedup 1.0000x reference)
import functools

import jax
import jax.numpy as jnp
from jax import lax
from jax.experimental import pallas as pl
from jax.experimental.pallas import tpu as pltpu
from jax.experimental.pallas import tpu_sc as plsc

EMB = 64
CH = 3
L = 16


def _sc_body(table_hbm, w_hbm, b_hbm, idx_hbm, out_hbm,
             idx_v, t_v, w_v, b_v, red_v, out_v):
    is_lead = (lax.axis_index("c") == 0) & (lax.axis_index("s") == 0)

    @pl.when(is_lead)
    def _():
        pltpu.sync_copy(idx_hbm, idx_v)
        pltpu.sync_copy(table_hbm, t_v)
        pltpu.sync_copy(w_hbm, w_v)
        pltpu.sync_copy(b_hbm, b_v)

        lanes = lax.iota(jnp.int32, L)
        zero_idx = jnp.zeros((L,), jnp.int32)

        idx_splat = plsc.load_gather(idx_v, [zero_idx])
        sel = [jnp.where(idx_splat == r, 1.0, 0.0) for r in range(CH)]
        row = []
        for k in range(EMB // L):
            chunk = sel[0] * t_v[0, pl.ds(k * L, L)]
            for r in range(1, CH):
                chunk = chunk + sel[r] * t_v[r, pl.ds(k * L, L)]
            row.append(chunk)

        bg = plsc.load_gather(b_v, [jnp.where(lanes < CH, lanes, 0)])
        acc = jnp.where(lanes < CH, bg, 0.0)

        for c in range(CH):
            dot = row[0] * w_v[c, pl.ds(0, L)]
            for k in range(1, EMB // L):
                dot = dot + row[k] * w_v[c, pl.ds(k * L, L)]
            for step in (8, 4, 2, 1):
                red_v[...] = dot
                dot = dot + plsc.load_gather(red_v, [(lanes + step) & (L - 1)])
            acc = acc + jnp.where(lanes == c, dot, 0.0)

        out_v[...] = acc
        pltpu.sync_copy(out_v.at[pl.ds(0, CH)], out_hbm)


@jax.jit
def _run(table, W, b, idx):
    mesh = plsc.VectorSubcoreMesh(core_axis_name="c", subcore_axis_name="s")
    f = functools.partial(
        pl.kernel,
        mesh=mesh,
        compiler_params=pltpu.CompilerParams(needs_layout_passes=False),
        out_type=jax.ShapeDtypeStruct((CH,), jnp.float32),
        scratch_types=[
            pltpu.VMEM((1,), jnp.int32),
            pltpu.VMEM((CH, EMB), jnp.float32),
            pltpu.VMEM((CH, EMB), jnp.float32),
            pltpu.VMEM((CH,), jnp.float32),
            pltpu.VMEM((L,), jnp.float32),
            pltpu.VMEM((L,), jnp.float32),
        ],
    )(_sc_body)
    return f(table, W, b, idx)


def kernel(table, W, b, idx):
    return _run(table, W, b, idx)


# speedup vs baseline: 1.0693x; 1.0693x over previous
import functools

import jax
import jax.numpy as jnp
from jax import lax
from jax.experimental import pallas as pl
from jax.experimental.pallas import tpu as pltpu
from jax.experimental.pallas import tpu_sc as plsc

EMB = 64
CH = 3
L = 16


def _sc_body(table_hbm, w_hbm, b_hbm, idx_hbm, out_hbm,
             idx_v, t_v, w_v, b_v, red_v, out_v, sem):
    is_lead = (lax.axis_index("c") == 0) & (lax.axis_index("s") == 0)

    @pl.when(is_lead)
    def _():
        c0 = pltpu.async_copy(idx_hbm, idx_v, sem)
        c1 = pltpu.async_copy(table_hbm, t_v, sem)
        c2 = pltpu.async_copy(w_hbm, w_v, sem)
        c3 = pltpu.async_copy(b_hbm, b_v, sem)
        c0.wait()
        c1.wait()
        c2.wait()
        c3.wait()

        lanes = lax.iota(jnp.int32, L)
        zero_idx = jnp.zeros((L,), jnp.int32)

        idx_splat = plsc.load_gather(idx_v, [zero_idx])
        sel = [jnp.where(idx_splat == r, 1.0, 0.0) for r in range(CH)]
        row = []
        for k in range(EMB // L):
            chunk = sel[0] * t_v[0, pl.ds(k * L, L)]
            for r in range(1, CH):
                chunk = chunk + sel[r] * t_v[r, pl.ds(k * L, L)]
            row.append(chunk)

        bg = plsc.load_gather(b_v, [jnp.where(lanes < CH, lanes, 0)])
        acc = jnp.where(lanes < CH, bg, 0.0)

        for c in range(CH):
            dot = row[0] * w_v[c, pl.ds(0, L)]
            for k in range(1, EMB // L):
                dot = dot + row[k] * w_v[c, pl.ds(k * L, L)]
            for step in (8, 4, 2, 1):
                red_v[...] = dot
                dot = dot + plsc.load_gather(red_v, [(lanes + step) & (L - 1)])
            acc = acc + jnp.where(lanes == c, dot, 0.0)

        out_v[...] = acc
        pltpu.sync_copy(out_v.at[pl.ds(0, CH)], out_hbm)


@jax.jit
def _run(table, W, b, idx):
    mesh = plsc.VectorSubcoreMesh(core_axis_name="c", subcore_axis_name="s")
    f = functools.partial(
        pl.kernel,
        mesh=mesh,
        compiler_params=pltpu.CompilerParams(needs_layout_passes=False),
        out_type=jax.ShapeDtypeStruct((CH,), jnp.float32),
        scratch_types=[
            pltpu.VMEM((1,), jnp.int32),
            pltpu.VMEM((CH, EMB), jnp.float32),
            pltpu.VMEM((CH, EMB), jnp.float32),
            pltpu.VMEM((CH,), jnp.float32),
            pltpu.VMEM((L,), jnp.float32),
            pltpu.VMEM((L,), jnp.float32),
            pltpu.SemaphoreType.DMA,
        ],
    )(_sc_body)
    return f(table, W, b, idx)


def kernel(table, W, b, idx):
    return _run(table, W, b, idx)


# speedup vs baseline: 1.1431x; 1.0691x over previous
import functools

import jax
import jax.numpy as jnp
from jax import lax
from jax.experimental import pallas as pl
from jax.experimental.pallas import tpu as pltpu
from jax.experimental.pallas import tpu_sc as plsc

EMB = 64
CH = 3
L = 16


def _sc_body(table_hbm, w_hbm, b_hbm, idx_hbm, out_hbm,
             idx_v, t_v, w_v, b_v, red_v, out_v, sem):
    is_lead = (lax.axis_index("c") == 0) & (lax.axis_index("s") == 0)

    @pl.when(is_lead)
    def _():
        c0 = pltpu.async_copy(idx_hbm, idx_v, sem)
        c1 = pltpu.async_copy(table_hbm, t_v, sem)
        c2 = pltpu.async_copy(w_hbm, w_v, sem)
        c3 = pltpu.async_copy(b_hbm, b_v, sem)
        c0.wait()
        c1.wait()
        c2.wait()
        c3.wait()

        lanes = lax.iota(jnp.int32, L)
        zero_idx = jnp.zeros((L,), jnp.int32)

        idx_splat = plsc.load_gather(idx_v, [zero_idx])
        sel = [jnp.where(idx_splat == r, 1.0, 0.0) for r in range(CH)]
        row = []
        for k in range(EMB // L):
            chunk = sel[0] * t_v[0, pl.ds(k * L, L)]
            for r in range(1, CH):
                chunk = chunk + sel[r] * t_v[r, pl.ds(k * L, L)]
            row.append(chunk)

        bg = plsc.load_gather(b_v, [jnp.where(lanes < CH, lanes, 0)])
        acc = jnp.where(lanes < CH, bg, 0.0)

        for c in range(CH):
            dot = row[0] * w_v[c, pl.ds(0, L)]
            for k in range(1, EMB // L):
                dot = dot + row[k] * w_v[c, pl.ds(k * L, L)]
            for step in (8, 4, 2, 1):
                red_v[...] = dot
                dot = dot + plsc.load_gather(red_v, [(lanes + step) & (L - 1)])
            acc = acc + jnp.where(lanes == c, dot, 0.0)

        out_v[...] = acc
        pltpu.sync_copy(out_v.at[pl.ds(0, CH)], out_hbm)


@jax.jit
def _run(table, W, b, idx):
    mesh = plsc.VectorSubcoreMesh(
        core_axis_name="c", subcore_axis_name="s", num_cores=1, num_subcores=1
    )
    f = functools.partial(
        pl.kernel,
        mesh=mesh,
        compiler_params=pltpu.CompilerParams(needs_layout_passes=False),
        out_type=jax.ShapeDtypeStruct((CH,), jnp.float32),
        scratch_types=[
            pltpu.VMEM((1,), jnp.int32),
            pltpu.VMEM((CH, EMB), jnp.float32),
            pltpu.VMEM((CH, EMB), jnp.float32),
            pltpu.VMEM((CH,), jnp.float32),
            pltpu.VMEM((L,), jnp.float32),
            pltpu.VMEM((L,), jnp.float32),
            pltpu.SemaphoreType.DMA,
        ],
    )(_sc_body)
    return f(table, W, b, idx)


def kernel(table, W, b, idx):
    return _run(table, W, b, idx)


# speedup vs baseline: 1.2248x; 1.0715x over previous
import functools

import jax
import jax.numpy as jnp
from jax import lax
from jax.experimental import pallas as pl
from jax.experimental.pallas import tpu as pltpu
from jax.experimental.pallas import tpu_sc as plsc

EMB = 64
CH = 3


def _scs_body(table_hbm, w_hbm, b_hbm, idx_hbm, out_hbm,
              t_s, w_s, b_s, idx_s, out_s, sem):
    c0 = pltpu.async_copy(idx_hbm, idx_s, sem)
    c1 = pltpu.async_copy(table_hbm, t_s, sem)
    c2 = pltpu.async_copy(w_hbm, w_s, sem)
    c3 = pltpu.async_copy(b_hbm, b_s, sem)
    c0.wait()
    c1.wait()
    c2.wait()
    c3.wait()

    i = idx_s[0]
    for c in range(CH):
        acc = b_s[c]
        for d in range(EMB):
            acc = acc + t_s[i, d] * w_s[c, d]
        out_s[c] = acc
    pltpu.sync_copy(out_s, out_hbm)


@jax.jit
def _run(table, W, b, idx):
    mesh = plsc.ScalarSubcoreMesh(axis_name="c", num_cores=1)
    f = functools.partial(
        pl.kernel,
        mesh=mesh,
        compiler_params=pltpu.CompilerParams(needs_layout_passes=False),
        out_type=jax.ShapeDtypeStruct((CH,), jnp.float32),
        scratch_types=[
            pltpu.SMEM((CH, EMB), jnp.float32),
            pltpu.SMEM((CH, EMB), jnp.float32),
            pltpu.SMEM((CH,), jnp.float32),
            pltpu.SMEM((1,), jnp.int32),
            pltpu.SMEM((CH,), jnp.float32),
            pltpu.SemaphoreType.DMA,
        ],
    )(_scs_body)
    return f(table, W, b, idx)


def kernel(table, W, b, idx):
    return _run(table, W, b, idx)


# speedup vs baseline: 1.2264x; 1.0013x over previous
import functools

import jax
import jax.numpy as jnp
from jax import lax
from jax.experimental import pallas as pl
from jax.experimental.pallas import tpu as pltpu
from jax.experimental.pallas import tpu_sc as plsc

EMB = 64
CH = 3


def _scs_body(table_hbm, w_hbm, b_hbm, idx_hbm, out_hbm,
              t_s, w_s, b_s, idx_s, out_s, sem):
    c0 = pltpu.async_copy(idx_hbm, idx_s, sem)
    c1 = pltpu.async_copy(table_hbm, t_s, sem)
    c2 = pltpu.async_copy(w_hbm, w_s, sem)
    c3 = pltpu.async_copy(b_hbm, b_s, sem)
    c0.wait()
    c1.wait()
    c2.wait()
    c3.wait()

    i = idx_s[0]
    acc = [b_s[c] for c in range(CH)]
    for d in range(EMB):
        t = t_s[i, d]
        for c in range(CH):
            acc[c] = acc[c] + t * w_s[c, d]
    for c in range(CH):
        out_s[c] = acc[c]
    pltpu.sync_copy(out_s, out_hbm)


@jax.jit
def _run(table, W, b, idx):
    mesh = plsc.ScalarSubcoreMesh(axis_name="c", num_cores=1)
    f = functools.partial(
        pl.kernel,
        mesh=mesh,
        compiler_params=pltpu.CompilerParams(needs_layout_passes=False),
        out_type=jax.ShapeDtypeStruct((CH,), jnp.float32),
        scratch_types=[
            pltpu.SMEM((CH, EMB), jnp.float32),
            pltpu.SMEM((CH, EMB), jnp.float32),
            pltpu.SMEM((CH,), jnp.float32),
            pltpu.SMEM((1,), jnp.int32),
            pltpu.SMEM((CH,), jnp.float32),
            pltpu.SemaphoreType.DMA,
        ],
    )(_scs_body)
    return f(table, W, b, idx)


def kernel(table, W, b, idx):
    return _run(table, W, b, idx)


# speedup vs baseline: 1.2586x; 1.0262x over previous
import functools

import jax
import jax.numpy as jnp
from jax import lax
from jax.experimental import pallas as pl
from jax.experimental.pallas import tpu as pltpu
from jax.experimental.pallas import tpu_sc as plsc

EMB = 64
CH = 3


def _scs_body(table_hbm, w_hbm, b_hbm, idx_hbm, out_hbm,
              t_s, w_s, b_s, idx_s, out_s, sem):
    c0 = pltpu.async_copy(idx_hbm, idx_s, sem)
    c1 = pltpu.async_copy(table_hbm, t_s, sem)
    c2 = pltpu.async_copy(w_hbm, w_s, sem)
    c3 = pltpu.async_copy(b_hbm, b_s, sem)
    c0.wait()
    c1.wait()
    c2.wait()
    c3.wait()

    i = idx_s[0]

    def step(d, acc):
        t = t_s[i, d]
        return (acc[0] + t * w_s[0, d],
                acc[1] + t * w_s[1, d],
                acc[2] + t * w_s[2, d])

    acc = lax.fori_loop(0, EMB, step, (b_s[0], b_s[1], b_s[2]))
    for c in range(CH):
        out_s[c] = acc[c]
    pltpu.sync_copy(out_s, out_hbm)


@jax.jit
def _run(table, W, b, idx):
    mesh = plsc.ScalarSubcoreMesh(axis_name="c", num_cores=1)
    f = functools.partial(
        pl.kernel,
        mesh=mesh,
        compiler_params=pltpu.CompilerParams(needs_layout_passes=False),
        out_type=jax.ShapeDtypeStruct((CH,), jnp.float32),
        scratch_types=[
            pltpu.SMEM((CH, EMB), jnp.float32),
            pltpu.SMEM((CH, EMB), jnp.float32),
            pltpu.SMEM((CH,), jnp.float32),
            pltpu.SMEM((1,), jnp.int32),
            pltpu.SMEM((CH,), jnp.float32),
            pltpu.SemaphoreType.DMA,
        ],
    )(_scs_body)
    return f(table, W, b, idx)


def kernel(table, W, b, idx):
    return _run(table, W, b, idx)
